# baseline (speedup 1.0000x reference)
import functools

import jax
import jax.numpy as jnp
from jax import lax
from jax.experimental import pallas as pl
from jax.experimental.pallas import tpu as pltpu
from jax.experimental.pallas import tpu_sc as plsc

_N_OUT = 1024
_NC = 2
_NS = 16
_NW = _NC * _NS
_RPW = _N_OUT // _NW
_CHUNK = 4


@functools.lru_cache(maxsize=None)
def _gather_call(n_rows, d):
    mesh = plsc.VectorSubcoreMesh(core_axis_name="c", subcore_axis_name="s")

    @functools.partial(
        pl.kernel,
        mesh=mesh,
        out_type=jax.ShapeDtypeStruct((_N_OUT, d), jnp.float32),
        scratch_types=[
            pltpu.VMEM((_RPW // _CHUNK, _CHUNK), jnp.int32),
            pltpu.VMEM((_CHUNK, d), jnp.float32),
            pltpu.SemaphoreType.DMA,
        ],
    )
    def k(x_hbm, idx_hbm, out_hbm, idx_v, buf_v, sem):
        wid = lax.axis_index("s") * _NC + lax.axis_index("c")
        base = wid * _RPW
        gbase = pl.multiple_of(base // _CHUNK, 8)
        pltpu.sync_copy(idx_hbm.at[pl.ds(gbase, _RPW // _CHUNK)], idx_v)
        for g in range(_RPW // _CHUNK):
            idx_chunk = idx_v.at[g]
            pltpu.async_copy(x_hbm.at[idx_chunk], buf_v, sem).wait()
            pltpu.sync_copy(buf_v, out_hbm.at[pl.ds(base + g * _CHUNK, _CHUNK)])

    return k


def kernel(x):
    n = x.shape[0]
    ck = jax.random.key(42)
    choices = jax.random.choice(ck, n, shape=(_N_OUT,), replace=False)
    choices = jnp.sort(choices).astype(jnp.int32).reshape(_N_OUT // _CHUNK, _CHUNK)
    d = x.shape[1] * x.shape[2] * x.shape[3]
    xf = x.reshape(n, d)
    out = _gather_call(n, d)(xf, choices)
    return out.reshape(_N_OUT, x.shape[1], x.shape[2], x.shape[3])
